# baseline (speedup 1.0000x reference)
import functools

import jax
import jax.numpy as jnp
from jax import lax
from jax.experimental import pallas as pl
from jax.experimental.pallas import tpu as pltpu
from jax.experimental.pallas import tpu_sc as plsc

_B, _C, _L, _N = 8, 128, 8192, 8192
_K = 5
_H = 64
_BN = _B * _N
_BL = _B * _L
_NW = 32
_PPW = _BN // _NW
_CHA = 128
_PB = 2048

import numpy as _np

_perm = _np.empty(_C, _np.int32)
for _q in range(_C // 32):
    for _i in range(16):
        _perm[32 * _q + 2 * _i] = 32 * _q + _i
        _perm[32 * _q + 2 * _i + 1] = 32 * _q + 16 + _i

_mesh = plsc.VectorSubcoreMesh(core_axis_name="c", subcore_axis_name="s")
_sc_params = pltpu.CompilerParams(needs_layout_passes=False)


def _worker_base():
    wid = lax.axis_index("s") * 2 + lax.axis_index("c")
    return wid * _PPW


def _anchor_ix(xv):
    return jnp.clip((xv + 1.0) * 0.5 * (_L - 1), 0.0, float(_L - 1))


@functools.partial(
    pl.kernel,
    out_type=(
        jax.ShapeDtypeStruct((_BN, _C), jnp.float32),
        jax.ShapeDtypeStruct((_BN, _C), jnp.float32),
    ),
    mesh=_mesh,
    scratch_types=[
        pltpu.VMEM((_PPW,), jnp.float32),
        pltpu.VMEM((_CHA,), jnp.int32),
        pltpu.VMEM((_CHA,), jnp.int32),
        pltpu.VMEM((_CHA, _C), jnp.float32),
        pltpu.VMEM((_CHA, _C), jnp.float32),
        pltpu.SemaphoreType.DMA,
        pltpu.SemaphoreType.DMA,
    ],
    compiler_params=_sc_params,
)
def _stage_a(coords_hbm, featT_hbm, f0_hbm, f1_hbm,
             coords_v, idx0_v, idx1_v, buf0_v, buf1_v, sem0, sem1):
    base = _worker_base()
    boff = (base // _N) * _L
    pltpu.sync_copy(coords_hbm.at[pl.ds(base, _PPW)], coords_v)

    def round_body(g, carry):
        rbase = g * _CHA
        for i in range(_CHA // 16):
            xv = coords_v[pl.ds(rbase + i * 16, 16)]
            ix = _anchor_ix(xv)
            x0 = ix.astype(jnp.int32)
            x1 = jnp.minimum(x0 + 1, _L - 1)
            idx0_v[pl.ds(i * 16, 16)] = x0 + boff
            idx1_v[pl.ds(i * 16, 16)] = x1 + boff
        cp0 = pltpu.async_copy(featT_hbm.at[idx0_v], buf0_v, sem0)
        cp1 = pltpu.async_copy(featT_hbm.at[idx1_v], buf1_v, sem1)
        cp0.wait()
        cp1.wait()
        pltpu.sync_copy(buf0_v, f0_hbm.at[pl.ds(base + rbase, _CHA)])
        pltpu.sync_copy(buf1_v, f1_hbm.at[pl.ds(base + rbase, _CHA)])
        return carry

    lax.fori_loop(0, _PPW // _CHA, round_body, 0)


def _dot(a, b):
    return lax.dot_general(a, b, (((1,), (0,)), ((), ())),
                           preferred_element_type=jnp.float32)


def _router_body(f0_ref, f1_ref, xr_ref, w1a_ref, w1cb_ref,
                 wre_ref, w2e_ref, gb_ref, ww_ref):
    xT = xr_ref[0]
    ixa = _anchor_ix(xT)
    x0f = jnp.floor(ixa)
    wa = ixa - x0f

    h0 = lax.dot_general(w1a_ref[...], f0_ref[...], (((1,), (1,)), ((), ())),
                         preferred_element_type=jnp.float32)
    h1 = lax.dot_general(w1a_ref[...], f1_ref[...], (((1,), (1,)), ((), ())),
                         preferred_element_type=jnp.float32)
    ones = jnp.ones_like(xT)
    x2 = jnp.concatenate([xT, ones], axis=0)
    h = h0 + wa * (h1 - h0) + _dot(w1cb_ref[...], x2)
    h = jnp.where(h >= 0, h, 0.2 * h)
    he = jnp.concatenate([h, ones], axis=0)
    h2 = h + _dot(wre_ref[...], he)
    h2 = jnp.where(h2 >= 0, h2, 0.2 * h2)
    h2e = jnp.concatenate([h2, ones], axis=0)
    rT = _dot(w2e_ref[...], h2e)

    offs = jnp.tanh(rT[:_K, :]) * (6.0 / _L)
    rw = rT[_K:, :]
    m = jnp.max(rw, axis=0, keepdims=True)
    e = jnp.exp(rw - m)
    dw = e / jnp.sum(e, axis=0, keepdims=True)

    xk = xT + offs
    ixk = _anchor_ix(xk)
    x0kf = jnp.floor(ixk)
    wk = ixk - x0kf
    x0k = x0kf.astype(jnp.int32)
    x1k = jnp.minimum(x0k + 1, _L - 1)

    x0a = x0f.astype(jnp.int32)
    wb = jnp.clip(x0a - 3, 0, _L - 8)
    par = wb & 1
    m0 = jnp.clip(x0k - wb, 0, 7) + par
    m1 = jnp.clip(x1k - wb, 0, 7) + par
    cw0 = dw * (1.0 - wk)
    cw1 = dw * wk

    rows = []
    for m in range(9):
        rows.append(
            jnp.sum(jnp.where(m0 == m, cw0, 0.0), axis=0, keepdims=True)
            + jnp.sum(jnp.where(m1 == m, cw1, 0.0), axis=0, keepdims=True))
    zero = jnp.zeros_like(rows[0])
    rows.extend([zero] * 7)
    wwT = jnp.concatenate(rows, axis=0)
    ci = lax.broadcasted_iota(jnp.int32, (16, 2 * _C), 1)
    expand = (2 * (ci // 32) + (ci % 2)
              == lax.broadcasted_iota(jnp.int32, (16, 2 * _C), 0)
              ).astype(jnp.float32)
    wwb = lax.dot_general(wwT, expand, (((0,), (0,)), ((), ())),
                          preferred_element_type=jnp.float32)

    b = pl.program_id(0) // (_N // _PB)
    gb_ref[0] = (wb + b * _L) >> 1
    ww_ref[...] = wwb.astype(jnp.bfloat16)


def _stage_b(f0, f1, xrow, w1a, w1cb, wre, w2e):
    grid = (_BN // _PB,)
    full = lambda shape: pl.BlockSpec(shape, lambda i: (0, 0))
    return pl.pallas_call(
        _router_body,
        grid=grid,
        in_specs=[
            pl.BlockSpec((_PB, _C), lambda i: (i, 0)),
            pl.BlockSpec((_PB, _C), lambda i: (i, 0)),
            pl.BlockSpec((1, 1, _PB), lambda i: (i, 0, 0)),
            full((_H, _C)),
            full((_H, 2)),
            full((_H, _H + 1)),
            full((2 * _K, _H + 1)),
        ],
        out_specs=[
            pl.BlockSpec((1, 1, _PB), lambda i: (i, 0, 0)),
            pl.BlockSpec((_PB, 2 * _C), lambda i: (i, 0)),
        ],
        out_shape=[
            jax.ShapeDtypeStruct((_BN // _PB, 1, _PB), jnp.int32),
            jax.ShapeDtypeStruct((_BN, 2 * _C), jnp.bfloat16),
        ],
    )(f0, f1, xrow, w1a, w1cb, wre, w2e)


_CHC = 16
_RC = _PPW // _CHC
_NS = 4


@functools.partial(
    pl.kernel,
    out_type=jax.ShapeDtypeStruct((_BN, _C), jnp.float32),
    mesh=_mesh,
    scratch_types=[
        pltpu.VMEM((_PPW,), jnp.int32),
        pltpu.VMEM((_NS * 128,), jnp.int32),
        pltpu.VMEM((_NS, 80, _C), jnp.int32),
        pltpu.VMEM((_NS, _CHC, _C), jnp.int32),
        pltpu.VMEM((_NS, _CHC, _C), jnp.float32),
        pltpu.SemaphoreType.DMA,
        pltpu.SemaphoreType.DMA,
        pltpu.SemaphoreType.DMA,
        pltpu.SemaphoreType.DMA,
        pltpu.SemaphoreType.DMA,
        pltpu.SemaphoreType.DMA,
        pltpu.SemaphoreType.DMA,
        pltpu.SemaphoreType.DMA,
    ],
    compiler_params=_sc_params,
)
def _stage_c(gb_hbm, wwb_hbm, featT_hbm, out_hbm,
             gb_v, idx_v, buf_v, ww_v, out_v,
             g0s, g1s, g2s, g3s, o0s, o1s, o2s, o3s):
    base = _worker_base()
    pltpu.sync_copy(gb_hbm.at[pl.ds(base, _PPW)], gb_v)
    lane = lax.iota(jnp.int32, 16)
    lane5 = lane * 5
    gsems = (g0s, g1s, g2s, g3s)
    osems = (o0s, o1s, o2s, o3s)

    def fire(g, s):
        p0 = g * _CHC
        gv = gb_v[pl.ds(p0, 16)]
        for m in range(5):
            plsc.store_scatter(
                idx_v, [lane5 + (s * 128 + m)],
                jnp.minimum(gv + m, _BL // 2 - 1))
        pltpu.async_copy(
            featT_hbm.at[idx_v.at[pl.ds(s * 128, 80)]], buf_v.at[s], gsems[s])
        pltpu.async_copy(
            wwb_hbm.at[pl.ds(base + p0, _CHC)], ww_v.at[s], gsems[s])

    def wait_slot(s):
        pltpu.make_async_copy(
            featT_hbm.at[idx_v.at[pl.ds(s * 128, 80)]], buf_v.at[s], gsems[s]
        ).wait()
        pltpu.make_async_copy(
            wwb_hbm.at[pl.ds(base, _CHC)], ww_v.at[s], gsems[s]).wait()

    def drain_out(s):
        pltpu.make_async_copy(
            out_v.at[s], out_hbm.at[pl.ds(base, _CHC)], osems[s]).wait()

    def compute(g, s):
        @pl.when(g >= _NS)
        def _():
            drain_out(s)

        def pbody(p2, c2):
            row = p2 * 5
            wp = [plsc.unpack(plsc.bitcast(ww_v[s, p2, pl.ds(u * 16, 16)],
                                           jnp.bfloat16),
                              format=plsc.PackFormat.INTERLEAVED,
                              preferred_element_type=jnp.float32)
                  for u in range(5)]
            for q in range(4):
                acc_a = jnp.zeros((16,), jnp.float32)
                acc_b = jnp.zeros((16,), jnp.float32)
                for u in range(5):
                    we, wo = wp[u]
                    va, vb = plsc.unpack(
                        plsc.bitcast(buf_v[s, row + u, pl.ds(q * 16, 16)],
                                     jnp.bfloat16),
                        format=plsc.PackFormat.INTERLEAVED,
                        preferred_element_type=jnp.float32)
                    acc_a = acc_a + va * we
                    acc_b = acc_b + vb * we
                    va, vb = plsc.unpack(
                        plsc.bitcast(buf_v[s, row + u, pl.ds(64 + q * 16, 16)],
                                     jnp.bfloat16),
                        format=plsc.PackFormat.INTERLEAVED,
                        preferred_element_type=jnp.float32)
                    acc_a = acc_a + va * wo
                    acc_b = acc_b + vb * wo
                out_v[s, p2, pl.ds(q * 32, 16)] = acc_a
                out_v[s, p2, pl.ds(q * 32 + 16, 16)] = acc_b
            return c2

        lax.fori_loop(0, _CHC, pbody, 0)
        pltpu.async_copy(
            out_v.at[s], out_hbm.at[pl.ds(base + g * _CHC, _CHC)], osems[s])

    for s in range(_NS):
        fire(s, s)

    def outer(t, carry):
        for s in range(_NS):
            g = _NS * t + s
            wait_slot(s)
            compute(g, s)

            @pl.when(g + _NS < _RC)
            def _():
                fire(g + _NS, s)

        return carry

    lax.fori_loop(0, _RC // _NS, outer, 0)
    for s in range(_NS):
        drain_out(s)


def kernel(feat_1d, coords_1d, W1, b1, Wr, br, W2, b2):
    assert feat_1d.shape == (_B, _C, _L) and coords_1d.shape == (_B, _N, 1)
    featT = jnp.transpose(feat_1d, (0, 2, 1)).reshape(_BL, _C)
    featP = lax.bitcast_convert_type(
        featT[:, _perm].astype(jnp.bfloat16).reshape(_BL // 2, _C, 2),
        jnp.int32)
    coords = coords_1d.reshape(_BN)
    f0, f1 = _stage_a(coords, featT)
    gb, ww = _stage_b(
        f0, f1, coords.reshape(_BN // _PB, 1, _PB),
        W1[:, :_C],
        jnp.stack([W1[:, _C], b1], axis=1),
        jnp.concatenate([Wr, br[:, None]], axis=1),
        jnp.concatenate([W2, b2[:, None]], axis=1),
    )
    ww_i = lax.bitcast_convert_type(
        ww.reshape(_BN, _C, 2), jnp.int32)
    out = _stage_c(gb.reshape(_BN), ww_i, featP)
    return out.reshape(_B, _N, _C)


# speedup vs baseline: 9.8439x; 9.8439x over previous
import functools

import jax
import jax.numpy as jnp
from jax import lax
from jax.experimental import pallas as pl
from jax.experimental.pallas import tpu as pltpu
from jax.experimental.pallas import tpu_sc as plsc

_B, _C, _L, _N = 8, 128, 8192, 8192
_K = 5
_H = 64
_BN = _B * _N
_BL = _B * _L
_NW = 32
_PPW = _BN // _NW
_CHA = 128
_PB = 2048

_mesh = plsc.VectorSubcoreMesh(core_axis_name="c", subcore_axis_name="s")
_sc_params = pltpu.CompilerParams(needs_layout_passes=False)


def _worker_base():
    wid = lax.axis_index("s") * 2 + lax.axis_index("c")
    return wid * _PPW


def _anchor_ix(xv):
    return jnp.clip((xv + 1.0) * 0.5 * (_L - 1), 0.0, float(_L - 1))


@functools.partial(
    pl.kernel,
    out_type=(
        jax.ShapeDtypeStruct((_BN, _C), jnp.float32),
        jax.ShapeDtypeStruct((_BN, _C), jnp.float32),
    ),
    mesh=_mesh,
    scratch_types=[
        pltpu.VMEM((_PPW,), jnp.float32),
        pltpu.VMEM((_CHA,), jnp.int32),
        pltpu.VMEM((_CHA,), jnp.int32),
        pltpu.VMEM((_CHA, _C), jnp.float32),
        pltpu.VMEM((_CHA, _C), jnp.float32),
        pltpu.SemaphoreType.DMA,
        pltpu.SemaphoreType.DMA,
    ],
    compiler_params=_sc_params,
)
def _stage_a(coords_hbm, featT_hbm, f0_hbm, f1_hbm,
             coords_v, idx0_v, idx1_v, buf0_v, buf1_v, sem0, sem1):
    base = _worker_base()
    boff = (base // _N) * _L
    pltpu.sync_copy(coords_hbm.at[pl.ds(base, _PPW)], coords_v)

    def round_body(g, carry):
        rbase = g * _CHA
        for i in range(_CHA // 16):
            xv = coords_v[pl.ds(rbase + i * 16, 16)]
            ix = _anchor_ix(xv)
            x0 = ix.astype(jnp.int32)
            x1 = jnp.minimum(x0 + 1, _L - 1)
            idx0_v[pl.ds(i * 16, 16)] = x0 + boff
            idx1_v[pl.ds(i * 16, 16)] = x1 + boff
        cp0 = pltpu.async_copy(featT_hbm.at[idx0_v], buf0_v, sem0)
        cp1 = pltpu.async_copy(featT_hbm.at[idx1_v], buf1_v, sem1)
        cp0.wait()
        cp1.wait()
        pltpu.sync_copy(buf0_v, f0_hbm.at[pl.ds(base + rbase, _CHA)])
        pltpu.sync_copy(buf1_v, f1_hbm.at[pl.ds(base + rbase, _CHA)])
        return carry

    lax.fori_loop(0, _PPW // _CHA, round_body, 0)


def _dot(a, b):
    return lax.dot_general(a, b, (((1,), (0,)), ((), ())),
                           preferred_element_type=jnp.float32)


def _router_body(f0_ref, f1_ref, xr_ref, w1a_ref, w1cb_ref,
                 wre_ref, w2e_ref, gb_ref, ww_ref):
    xT = xr_ref[0]
    ixa = _anchor_ix(xT)
    x0f = jnp.floor(ixa)
    wa = ixa - x0f

    h0 = lax.dot_general(w1a_ref[...], f0_ref[...], (((1,), (1,)), ((), ())),
                         preferred_element_type=jnp.float32)
    h1 = lax.dot_general(w1a_ref[...], f1_ref[...], (((1,), (1,)), ((), ())),
                         preferred_element_type=jnp.float32)
    ones = jnp.ones_like(xT)
    x2 = jnp.concatenate([xT, ones], axis=0)
    h = h0 + wa * (h1 - h0) + _dot(w1cb_ref[...], x2)
    h = jnp.where(h >= 0, h, 0.2 * h)
    he = jnp.concatenate([h, ones], axis=0)
    h2 = h + _dot(wre_ref[...], he)
    h2 = jnp.where(h2 >= 0, h2, 0.2 * h2)
    h2e = jnp.concatenate([h2, ones], axis=0)
    rT = _dot(w2e_ref[...], h2e)

    offs = jnp.tanh(rT[:_K, :]) * (6.0 / _L)
    rw = rT[_K:, :]
    m = jnp.max(rw, axis=0, keepdims=True)
    e = jnp.exp(rw - m)
    dw = e / jnp.sum(e, axis=0, keepdims=True)

    xk = xT + offs
    ixk = _anchor_ix(xk)
    x0kf = jnp.floor(ixk)
    wk = ixk - x0kf
    x0k = x0kf.astype(jnp.int32)
    x1k = jnp.minimum(x0k + 1, _L - 1)

    x0a = x0f.astype(jnp.int32)
    wb = jnp.clip(x0a - 3, 0, _L - 8)
    p0 = jnp.clip(x0k - wb, 0, 7)
    p1 = jnp.clip(x1k - wb, 0, 7)
    cw0 = dw * (1.0 - wk)
    cw1 = dw * wk

    rows = []
    for j in range(8):
        rows.append(
            jnp.sum(jnp.where(p0 == j, cw0, 0.0), axis=0, keepdims=True)
            + jnp.sum(jnp.where(p1 == j, cw1, 0.0), axis=0, keepdims=True))
    wwT = jnp.concatenate(rows, axis=0)
    expand = (lax.broadcasted_iota(jnp.int32, (8, _C), 1) // 16
              == lax.broadcasted_iota(jnp.int32, (8, _C), 0)).astype(jnp.float32)
    wwb = lax.dot_general(wwT, expand, (((0,), (0,)), ((), ())),
                          preferred_element_type=jnp.float32)

    b = pl.program_id(0) // (_N // _PB)
    gb_ref[0] = wb + b * _L
    ww_ref[...] = wwb


def _stage_b(f0, f1, xrow, w1a, w1cb, wre, w2e):
    grid = (_BN // _PB,)
    full = lambda shape: pl.BlockSpec(shape, lambda i: (0, 0))
    return pl.pallas_call(
        _router_body,
        grid=grid,
        in_specs=[
            pl.BlockSpec((_PB, _C), lambda i: (i, 0)),
            pl.BlockSpec((_PB, _C), lambda i: (i, 0)),
            pl.BlockSpec((1, 1, _PB), lambda i: (i, 0, 0)),
            full((_H, _C)),
            full((_H, 2)),
            full((_H, _H + 1)),
            full((2 * _K, _H + 1)),
        ],
        out_specs=[
            pl.BlockSpec((1, 1, _PB), lambda i: (i, 0, 0)),
            pl.BlockSpec((_PB, _C), lambda i: (i, 0)),
        ],
        out_shape=[
            jax.ShapeDtypeStruct((_BN // _PB, 1, _PB), jnp.int32),
            jax.ShapeDtypeStruct((_BN, _C), jnp.float32),
        ],
    )(f0, f1, xrow, w1a, w1cb, wre, w2e)


_CHC = 16
_RC = _PPW // _CHC
_NS = 4


@functools.partial(
    pl.kernel,
    out_type=jax.ShapeDtypeStruct((_BN, _C), jnp.float32),
    mesh=_mesh,
    scratch_types=[
        pltpu.VMEM((_PPW,), jnp.int32),
        pltpu.VMEM((_NS * 128,), jnp.int32),
        pltpu.VMEM((_NS, 128, _C), jnp.float32),
        pltpu.VMEM((_NS, _CHC, _C), jnp.float32),
        pltpu.VMEM((_NS, _CHC, _C), jnp.float32),
        pltpu.SemaphoreType.DMA,
        pltpu.SemaphoreType.DMA,
        pltpu.SemaphoreType.DMA,
        pltpu.SemaphoreType.DMA,
        pltpu.SemaphoreType.DMA,
        pltpu.SemaphoreType.DMA,
        pltpu.SemaphoreType.DMA,
        pltpu.SemaphoreType.DMA,
    ],
    compiler_params=_sc_params,
)
def _stage_c(gb_hbm, wwb_hbm, featT_hbm, out_hbm,
             gb_v, idx_v, buf_v, ww_v, out_v,
             g0s, g1s, g2s, g3s, o0s, o1s, o2s, o3s):
    base = _worker_base()
    pltpu.sync_copy(gb_hbm.at[pl.ds(base, _PPW)], gb_v)
    lane = lax.iota(jnp.int32, 16)
    lane8 = lane * 8
    gsems = (g0s, g1s, g2s, g3s)
    osems = (o0s, o1s, o2s, o3s)

    def fire(g, s):
        p0 = g * _CHC
        gv = gb_v[pl.ds(p0, 16)]
        for j in range(8):
            plsc.store_scatter(idx_v, [lane8 + (s * 128 + j)], gv + j)
        pltpu.async_copy(
            featT_hbm.at[idx_v.at[pl.ds(s * 128, 128)]], buf_v.at[s], gsems[s])
        pltpu.async_copy(
            wwb_hbm.at[pl.ds(base + p0, _CHC)], ww_v.at[s], gsems[s])

    def wait_slot(s):
        pltpu.make_async_copy(
            featT_hbm.at[idx_v.at[pl.ds(s * 128, 128)]], buf_v.at[s], gsems[s]
        ).wait()
        pltpu.make_async_copy(
            wwb_hbm.at[pl.ds(base, _CHC)], ww_v.at[s], gsems[s]).wait()

    def drain_out(s):
        pltpu.make_async_copy(
            out_v.at[s], out_hbm.at[pl.ds(base, _CHC)], osems[s]).wait()

    def compute(g, s):
        @pl.when(g >= _NS)
        def _():
            drain_out(s)

        def pbody(p2, c2):
            row = p2 * 8
            wvs = [ww_v[s, p2, pl.ds(j * 16, 16)] for j in range(8)]
            for cv in range(8):
                sl = pl.ds(cv * 16, 16)
                t0 = wvs[0] * buf_v[s, row, sl] + wvs[1] * buf_v[s, row + 1, sl]
                t1 = wvs[2] * buf_v[s, row + 2, sl] + wvs[3] * buf_v[s, row + 3, sl]
                t2 = wvs[4] * buf_v[s, row + 4, sl] + wvs[5] * buf_v[s, row + 5, sl]
                t3 = wvs[6] * buf_v[s, row + 6, sl] + wvs[7] * buf_v[s, row + 7, sl]
                out_v[s, p2, sl] = (t0 + t1) + (t2 + t3)
            return c2

        lax.fori_loop(0, _CHC, pbody, 0)
        pltpu.async_copy(
            out_v.at[s], out_hbm.at[pl.ds(base + g * _CHC, _CHC)], osems[s])

    for s in range(_NS):
        fire(s, s)

    def outer(t, carry):
        for s in range(_NS):
            g = _NS * t + s
            wait_slot(s)
            compute(g, s)

            @pl.when(g + _NS < _RC)
            def _():
                fire(g + _NS, s)

        return carry

    lax.fori_loop(0, _RC // _NS, outer, 0)
    for s in range(_NS):
        drain_out(s)


def kernel(feat_1d, coords_1d, W1, b1, Wr, br, W2, b2):
    assert feat_1d.shape == (_B, _C, _L) and coords_1d.shape == (_B, _N, 1)
    featT = jnp.transpose(feat_1d, (0, 2, 1)).reshape(_BL, _C)
    coords = coords_1d.reshape(_BN)
    f0, f1 = _stage_a(coords, featT)
    gb, ww = _stage_b(
        f0, f1, coords.reshape(_BN // _PB, 1, _PB),
        W1[:, :_C],
        jnp.stack([W1[:, _C], b1], axis=1),
        jnp.concatenate([Wr, br[:, None]], axis=1),
        jnp.concatenate([W2, b2[:, None]], axis=1),
    )
    out = _stage_c(gb.reshape(_BN), ww, featT)
    return out.reshape(_B, _N, _C)
